# baseline (speedup 1.0000x reference)
import functools

import jax
import jax.numpy as jnp
from jax import lax
from jax.experimental import pallas as pl
from jax.experimental.pallas import tpu as pltpu
from jax.experimental.pallas import tpu_sc as plsc

BATCH = 16384
EMB = 64
EMBP = 128
N_G = 21
N_I = 5
BLK = 4096


@functools.lru_cache(maxsize=None)
def _make_sc_gather(V, D, B):
    info = plsc.get_sparse_core_info()
    NC, NS = info.num_cores, info.num_subcores
    NW = NC * NS
    b_per_w = B // NW
    mesh = plsc.VectorSubcoreMesh(core_axis_name="c", subcore_axis_name="s")

    NCHUNK = 4
    CH = b_per_w // NCHUNK

    @functools.partial(
        pl.kernel,
        mesh=mesh,
        compiler_params=pltpu.CompilerParams(use_tc_tiling_on_sc=True),
        out_type=jax.ShapeDtypeStruct((B, D), jnp.float32),
        scratch_types=[
            pltpu.VMEM((b_per_w,), jnp.int32),
            pltpu.VMEM((NCHUNK, CH, D), jnp.float32),
            pltpu.SemaphoreType.DMA,
            pltpu.SemaphoreType.DMA,
        ],
    )
    def gather_k(table_hbm, idx_hbm, out_hbm, idx_v, rows_v, sem0, sem1):
        wid = lax.axis_index("s") * NC + lax.axis_index("c")
        base = wid * b_per_w
        sems = (sem0, sem1)
        pltpu.sync_copy(idx_hbm.at[pl.ds(base, b_per_w)], idx_v)
        cps = [
            pltpu.async_copy(table_hbm.at[idx_v.at[pl.ds(c * CH, CH)]],
                             rows_v.at[c], sems[c % 2])
            for c in range(2)
        ]
        for c in range(NCHUNK):
            cps[c].wait()
            if c + 2 < NCHUNK:
                cps.append(
                    pltpu.async_copy(
                        table_hbm.at[idx_v.at[pl.ds((c + 2) * CH, CH)]],
                        rows_v.at[c + 2], sems[c % 2]))
            pltpu.sync_copy(rows_v.at[c],
                            out_hbm.at[pl.ds(base + c * CH, CH)])

    return gather_k


def _mlp_body(g_ref, i_ref, emb_ref, w1a_ref, w1b_ref, b1_ref, w2_ref, b2_ref,
              o_ref):
    g = g_ref[...].reshape(1, BLK)
    i = i_ref[...].reshape(1, BLK)
    row = lax.broadcasted_iota(jnp.int32, (EMB, BLK), 0)
    ohT = ((row == g) | (row == i)).astype(jnp.float32)
    h = jnp.dot(emb_ref[...], w1a_ref[...], preferred_element_type=jnp.float32)
    h = h + lax.dot_general(ohT, w1b_ref[...], (((0,), (0,)), ((), ())),
                            preferred_element_type=jnp.float32)
    h = jnp.maximum(h + b1_ref[...], 0.0)
    o_ref[...] = (lax.dot_general(w2_ref[...], h, (((0,), (1,)), ((), ())),
                                  preferred_element_type=jnp.float32)
                  + b2_ref[...])


def _mlp(g2, i2, emb, W1a, W1b, b1r, W2, b2r):
    grid = BATCH // BLK
    full = lambda j: (0, 0)
    return pl.pallas_call(
        _mlp_body,
        grid=(grid,),
        in_specs=[
            pl.BlockSpec((1, 1, BLK), lambda j: (j, 0, 0)),
            pl.BlockSpec((1, 1, BLK), lambda j: (j, 0, 0)),
            pl.BlockSpec((BLK, EMBP), lambda j: (j, 0)),
            pl.BlockSpec((EMBP, EMB), full),
            pl.BlockSpec((EMB, EMB), full),
            pl.BlockSpec((1, EMB), full),
            pl.BlockSpec((EMB, EMB), full),
            pl.BlockSpec((EMB, 1), full),
        ],
        out_specs=pl.BlockSpec((EMB, BLK), lambda j: (0, j)),
        out_shape=jax.ShapeDtypeStruct((EMB, BATCH), jnp.float32),
    )(g2, i2, emb, W1a, W1b, b1r, W2, b2r)


def kernel(article_id, garment_group_name, index_group_name, table, W1, b1,
           W2, b2):
    aid = article_id.astype(jnp.int32)
    v_pad = (-table.shape[0]) % 8
    table_p = jnp.pad(table, ((0, v_pad), (0, EMBP - EMB)))
    emb = _make_sc_gather(table_p.shape[0], EMBP, BATCH)(table_p, aid)
    grid = BATCH // BLK
    g0 = garment_group_name.astype(jnp.int32)
    g2 = jnp.where(g0 >= N_G, -1, g0).reshape(grid, 1, BLK)
    i2 = (index_group_name.astype(jnp.int32) + N_G).reshape(grid, 1, BLK)
    W1a = jnp.pad(W1[:EMB], ((0, EMBP - EMB), (0, 0)))
    W1b = jnp.pad(W1[EMB:], ((0, EMB - (N_G + N_I)), (0, 0)))
    out_t = _mlp(g2, i2, emb, W1a, W1b, b1.reshape(1, EMB), W2,
                 b2.reshape(EMB, 1))
    return out_t.T


# speedup vs baseline: 1.0276x; 1.0276x over previous
import functools

import jax
import jax.numpy as jnp
from jax import lax
from jax.experimental import pallas as pl
from jax.experimental.pallas import tpu as pltpu
from jax.experimental.pallas import tpu_sc as plsc

BATCH = 16384
EMB = 64
EMBP = 128
N_G = 21
N_I = 5
BLK = 8192


@functools.lru_cache(maxsize=None)
def _make_sc_gather(V, D, B, dtype=jnp.float32):
    info = plsc.get_sparse_core_info()
    NC, NS = info.num_cores, info.num_subcores
    NW = NC * NS
    b_per_w = B // NW
    mesh = plsc.VectorSubcoreMesh(core_axis_name="c", subcore_axis_name="s")

    NCHUNK = 4
    CH = b_per_w // NCHUNK

    @functools.partial(
        pl.kernel,
        mesh=mesh,
        compiler_params=pltpu.CompilerParams(use_tc_tiling_on_sc=True),
        out_type=jax.ShapeDtypeStruct((B, D), dtype),
        scratch_types=[
            pltpu.VMEM((b_per_w,), jnp.int32),
            pltpu.VMEM((NCHUNK, CH, D), dtype),
            pltpu.SemaphoreType.DMA,
            pltpu.SemaphoreType.DMA,
        ],
    )
    def gather_k(table_hbm, idx_hbm, out_hbm, idx_v, rows_v, sem0, sem1):
        wid = lax.axis_index("s") * NC + lax.axis_index("c")
        base = wid * b_per_w
        sems = (sem0, sem1)
        pltpu.sync_copy(idx_hbm.at[pl.ds(base, b_per_w)], idx_v)
        cps = [
            pltpu.async_copy(table_hbm.at[idx_v.at[pl.ds(c * CH, CH)]],
                             rows_v.at[c], sems[c % 2])
            for c in range(2)
        ]
        for c in range(NCHUNK):
            cps[c].wait()
            if c + 2 < NCHUNK:
                cps.append(
                    pltpu.async_copy(
                        table_hbm.at[idx_v.at[pl.ds((c + 2) * CH, CH)]],
                        rows_v.at[c + 2], sems[c % 2]))
            pltpu.sync_copy(rows_v.at[c],
                            out_hbm.at[pl.ds(base + c * CH, CH)])

    return gather_k


def _mlp_body(g_ref, i_ref, emb_ref, w1a_ref, w1b_ref, b1_ref, w2_ref, b2_ref,
              o_ref):
    g = g_ref[...].reshape(1, BLK)
    i = i_ref[...].reshape(1, BLK)
    row = lax.broadcasted_iota(jnp.int32, (EMB, BLK), 0)
    ohT = ((row == g) | (row == i)).astype(jnp.float32)
    h = jnp.dot(emb_ref[...], w1a_ref[...], preferred_element_type=jnp.float32)
    h = h + lax.dot_general(ohT, w1b_ref[...], (((0,), (0,)), ((), ())),
                            preferred_element_type=jnp.float32)
    h = jnp.maximum(h + b1_ref[...], 0.0)
    o_ref[...] = (lax.dot_general(w2_ref[...], h, (((0,), (1,)), ((), ())),
                                  preferred_element_type=jnp.float32)
                  + b2_ref[...])


def _mlp(g2, i2, emb, W1a, W1b, b1r, W2, b2r):
    grid = BATCH // BLK
    full = lambda j: (0, 0)
    return pl.pallas_call(
        _mlp_body,
        grid=(grid,),
        in_specs=[
            pl.BlockSpec((1, 1, BLK), lambda j: (j, 0, 0)),
            pl.BlockSpec((1, 1, BLK), lambda j: (j, 0, 0)),
            pl.BlockSpec((BLK, EMBP), lambda j: (j, 0)),
            pl.BlockSpec((EMBP, EMB), full),
            pl.BlockSpec((EMB, EMB), full),
            pl.BlockSpec((1, EMB), full),
            pl.BlockSpec((EMB, EMB), full),
            pl.BlockSpec((EMB, 1), full),
        ],
        out_specs=pl.BlockSpec((EMB, BLK), lambda j: (0, j)),
        out_shape=jax.ShapeDtypeStruct((EMB, BATCH), jnp.float32),
    )(g2, i2, emb, W1a, W1b, b1r, W2, b2r)


def kernel(article_id, garment_group_name, index_group_name, table, W1, b1,
           W2, b2):
    aid = article_id.astype(jnp.int32)
    v_pad = (-table.shape[0]) % 8
    table_p = jnp.pad(table, ((0, v_pad), (0, EMBP - EMB)))
    emb = _make_sc_gather(table_p.shape[0], EMBP, BATCH)(table_p, aid)
    grid = BATCH // BLK
    g0 = garment_group_name.astype(jnp.int32)
    g2 = jnp.where(g0 >= N_G, -1, g0).reshape(grid, 1, BLK)
    i2 = (index_group_name.astype(jnp.int32) + N_G).reshape(grid, 1, BLK)
    W1a = jnp.pad(W1[:EMB], ((0, EMBP - EMB), (0, 0)))
    W1b = jnp.pad(W1[EMB:], ((0, EMB - (N_G + N_I)), (0, 0)))
    out_t = _mlp(g2, i2, emb, W1a, W1b, b1.reshape(1, EMB), W2,
                 b2.reshape(EMB, 1))
    return out_t.T
